# baseline (speedup 1.0000x reference)
import functools

import jax
import jax.numpy as jnp
from jax import lax
from jax.experimental import pallas as pl
from jax.experimental.pallas import tpu as pltpu
from jax.experimental.pallas import tpu_sc as plsc

_B, _H, _S, _D = 4, 16, 4096, 128
_N = _B * _H * _S
_NW = 32
_PER = _N // _NW
_ROWS = _PER // 128


@jax.jit
def _sc_select(x1, idxv):
    mesh = plsc.VectorSubcoreMesh(core_axis_name="c", subcore_axis_name="s")

    @functools.partial(
        pl.kernel,
        mesh=mesh,
        out_type=jax.ShapeDtypeStruct((_B, _H, _S), jnp.float32),
        scratch_types=[
            pltpu.VMEM((16,), jnp.int32),
            pltpu.VMEM((_ROWS, 128), jnp.int32),
            pltpu.VMEM((2, _S), jnp.float32),
            pltpu.SemaphoreType.DMA,
        ],
        compiler_params=pltpu.CompilerParams(use_tc_tiling_on_sc=False),
    )
    def k(x_hbm, idx_hbm, out_hbm, idx_v, gidx, buf, sem):
        wid = lax.axis_index("s") * 2 + lax.axis_index("c")
        pltpu.sync_copy(idx_hbm, idx_v)
        vidx = idx_v[...]
        lane = lax.iota(jnp.int32, 16) * _D

        b = wid // 8
        w8 = wid % 8
        hb = w8 // 4
        sb0 = (w8 % 4) * 8

        copies = []
        for j in range(_ROWS):
            s0 = (wid * _PER + j * 128) * _D + vidx
            for kk in range(8):
                gidx[j, pl.ds(kk * 16, 16)] = s0 + (kk * 16 * _D) + lane
            copies.append(
                pltpu.async_copy(
                    x_hbm.at[gidx.at[j]],
                    buf.at[j // 32, pl.ds((j % 32) * 128, 128)],
                    sem,
                )
            )
        for c in copies:
            c.wait()
        pltpu.sync_copy(buf, out_hbm.at[b, pl.ds(w8 * 2, 2)])

    return k(x1, idxv)


def kernel(x, dim, index):
    idx = (jnp.asarray(index) + jnp.asarray(dim) - 3).astype(jnp.int32)
    x1 = x.reshape(_N * _D)
    idxv = jnp.full((16,), idx, jnp.int32)
    return _sc_select(x1, idxv)


# speedup vs baseline: 1.0558x; 1.0558x over previous
import functools

import jax
import jax.numpy as jnp
from jax import lax
from jax.experimental import pallas as pl
from jax.experimental.pallas import tpu as pltpu
from jax.experimental.pallas import tpu_sc as plsc

_B, _H, _S, _D = 4, 16, 4096, 128
_N = _B * _H * _S
_NW = 32
_PER = _N // _NW
_ROWS = _PER // 128


@jax.jit
def _sc_select(x1, idxv):
    mesh = plsc.VectorSubcoreMesh(core_axis_name="c", subcore_axis_name="s")

    @functools.partial(
        pl.kernel,
        mesh=mesh,
        out_type=jax.ShapeDtypeStruct((_B, _H, _S), jnp.float32),
        scratch_types=[
            pltpu.VMEM((16,), jnp.int32),
            pltpu.VMEM((_ROWS, 128), jnp.int32),
            pltpu.VMEM((8, 1024), jnp.float32),
            pltpu.SemaphoreType.DMA,
        ],
        compiler_params=pltpu.CompilerParams(use_tc_tiling_on_sc=True),
    )
    def k(x_hbm, idx_hbm, out_hbm, idx_v, gidx, buf, sem):
        wid = lax.axis_index("s") * 2 + lax.axis_index("c")
        pltpu.sync_copy(idx_hbm, idx_v)
        vidx = idx_v[...]
        lane = lax.iota(jnp.int32, 16) * _D

        b = wid // 8
        w8 = wid % 8
        h0 = (w8 // 4) * 8
        s0 = (w8 % 4) * 1024

        copies = []
        for j in range(_ROWS):
            r, c = j // 8, j % 8
            base = ((b * _H + h0 + r) * _S + s0 + c * 128) * _D + vidx
            for kk in range(8):
                gidx[j, pl.ds(kk * 16, 16)] = base + (kk * 16 * _D) + lane
            copies.append(
                pltpu.async_copy(
                    x_hbm.at[gidx.at[j]],
                    buf.at[r, pl.ds(c * 128, 128)],
                    sem,
                )
            )
        for c in copies:
            c.wait()
        pltpu.sync_copy(buf, out_hbm.at[b, pl.ds(h0, 8), pl.ds(s0, 1024)])

    return k(x1, idxv)


def kernel(x, dim, index):
    idx = (jnp.asarray(index) + jnp.asarray(dim) - 3).astype(jnp.int32)
    x1 = x.reshape(_N * _D)
    idxv = jnp.full((16,), idx, jnp.int32)
    return _sc_select(x1, idxv)


# speedup vs baseline: 1.1266x; 1.0670x over previous
import functools

import jax
import jax.numpy as jnp
from jax import lax
from jax.experimental import pallas as pl
from jax.experimental.pallas import tpu as pltpu
from jax.experimental.pallas import tpu_sc as plsc

_B, _H, _S, _D = 4, 16, 4096, 128
_N = _B * _H * _S
_NW = 32
_PER = _N // _NW
_ROWS = _PER // 128


@jax.jit
def _sc_select(x1, idxv):
    mesh = plsc.VectorSubcoreMesh(core_axis_name="c", subcore_axis_name="s")

    @functools.partial(
        pl.kernel,
        mesh=mesh,
        out_type=jax.ShapeDtypeStruct((_B, _H, _S), jnp.float32),
        scratch_types=[
            pltpu.VMEM((16,), jnp.int32),
            pltpu.VMEM((_ROWS, 128), jnp.int32),
            pltpu.VMEM((8, 1024), jnp.float32),
            pltpu.SemaphoreType.DMA,
        ],
        compiler_params=pltpu.CompilerParams(use_tc_tiling_on_sc=True),
    )
    def k(x_hbm, idx_hbm, out_hbm, idx_v, gidx, buf, sem):
        wid = lax.axis_index("s") * 2 + lax.axis_index("c")
        pltpu.sync_copy(idx_hbm, idx_v)
        vidx = idx_v[...]
        lane = lax.iota(jnp.int32, 16) * _D

        b = wid // 8
        w8 = wid % 8
        h0 = (w8 // 4) * 8
        s0 = (w8 % 4) * 1024

        def fill_and_fire(j, carry):
            r, c = j // 8, j % 8
            base = ((b * _H + h0 + r) * _S + s0 + c * 128) * _D + vidx
            for kk in range(8):
                gidx[j, pl.ds(kk * 16, 16)] = base + (kk * 16 * _D) + lane
            pltpu.async_copy(
                x_hbm.at[gidx.at[j]],
                buf.at[r, pl.ds(c * 128, 128)],
                sem,
            )
            return carry

        lax.fori_loop(0, _ROWS, fill_and_fire, 0)
        out_blk = out_hbm.at[b, pl.ds(h0, 8), pl.ds(s0, 1024)]
        pltpu.make_async_copy(out_blk, buf, sem).wait()
        pltpu.sync_copy(buf, out_blk)

    return k(x1, idxv)


def kernel(x, dim, index):
    idx = (jnp.asarray(index) + jnp.asarray(dim) - 3).astype(jnp.int32)
    x1 = x.reshape(_N * _D)
    idxv = jnp.full((16,), idx, jnp.int32)
    return _sc_select(x1, idxv)


# speedup vs baseline: 1.1278x; 1.0011x over previous
import functools

import jax
import jax.numpy as jnp
from jax import lax
from jax.experimental import pallas as pl
from jax.experimental.pallas import tpu as pltpu
from jax.experimental.pallas import tpu_sc as plsc

_B, _H, _S, _D = 4, 16, 4096, 128
_N = _B * _H * _S
_NW = 32
_PER = _N // _NW
_ROWS = _PER // 128


@jax.jit
def _sc_select(x1, idxv):
    mesh = plsc.VectorSubcoreMesh(core_axis_name="c", subcore_axis_name="s")

    @functools.partial(
        pl.kernel,
        mesh=mesh,
        out_type=jax.ShapeDtypeStruct((_B, _H, _S), jnp.float32),
        scratch_types=[
            pltpu.VMEM((16,), jnp.int32),
            pltpu.VMEM((_ROWS, 128), jnp.int32),
            pltpu.VMEM((8, 1024), jnp.float32),
            pltpu.SemaphoreType.DMA,
        ],
        compiler_params=pltpu.CompilerParams(
            use_tc_tiling_on_sc=True,
            disable_bounds_checks=True,
            disable_semaphore_checks=True,
            skip_device_barrier=True,
        ),
    )
    def k(x_hbm, idx_hbm, out_hbm, idx_v, gidx, buf, sem):
        wid = lax.axis_index("s") * 2 + lax.axis_index("c")
        pltpu.sync_copy(idx_hbm, idx_v)
        vidx = idx_v[...]
        lane = lax.iota(jnp.int32, 16) * _D

        b = wid // 8
        w8 = wid % 8
        h0 = (w8 // 4) * 8
        s0 = (w8 % 4) * 1024

        def fill_and_fire(j, carry):
            r, c = j // 8, j % 8
            base = ((b * _H + h0 + r) * _S + s0 + c * 128) * _D + vidx
            for kk in range(8):
                gidx[j, pl.ds(kk * 16, 16)] = base + (kk * 16 * _D) + lane
            pltpu.async_copy(
                x_hbm.at[gidx.at[j]],
                buf.at[r, pl.ds(c * 128, 128)],
                sem,
            )
            return carry

        lax.fori_loop(0, _ROWS, fill_and_fire, 0)
        out_blk = out_hbm.at[b, pl.ds(h0, 8), pl.ds(s0, 1024)]
        pltpu.make_async_copy(out_blk, buf, sem).wait()
        pltpu.sync_copy(buf, out_blk)

    return k(x1, idxv)


def kernel(x, dim, index):
    idx = (jnp.asarray(index) + jnp.asarray(dim) - 3).astype(jnp.int32)
    x1 = x.reshape(_N * _D)
    idxv = jnp.full((16,), idx, jnp.int32)
    return _sc_select(x1, idxv)
